# Initial kernel scaffold; baseline (speedup 1.0000x reference)
import functools

import jax
import jax.numpy as jnp
from jax import lax
from jax.experimental import pallas as pl
from jax.experimental.pallas import tpu as pltpu
from jax.experimental.pallas import tpu_sc as plsc

_BN_EPS = 1e-3
_NUM_BUCKETS = 11
_BOUNDS = tuple(float(v) for v in range(1, 100, 10))
_U_STRIDE = 128
_B = 16384
_D_OUT = 16
_NC, _NS = 2, 16
_NW = _NC * _NS
_BPW = _B // _NW
_CHUNK = 128
_N_CHUNKS = _BPW // _CHUNK


def _mlp_table_body(uid_ref, age_ref, s1a_ref, t1a_ref, s1b_ref, t1b_ref,
                    w1a_ref, w1b_ref, b1_ref, s2_ref, t2_ref, w2_ref, b2_ref,
                    f_ref):
    xu = uid_ref[...] * s1a_ref[...] + t1a_ref[...]
    xa = age_ref[...] * s1b_ref[...] + t1b_ref[...]
    h = jnp.dot(xu, w1a_ref[...], preferred_element_type=jnp.float32)
    h = h + jnp.dot(xa, w1b_ref[...], preferred_element_type=jnp.float32)
    h = jnp.maximum(h + b1_ref[...], 0.0)
    h = h * s2_ref[...] + t2_ref[...]
    f = jnp.dot(h, w2_ref[...], preferred_element_type=jnp.float32) + b2_ref[...]
    f_ref[...] = jnp.maximum(f, 0.0)


def _build_f_table(uid_pad, age_table, s1a, t1a, s1b, t1b, w1a, w1b, b1,
                   s2, t2, w2, b2):
    const = lambda shape: pl.BlockSpec(shape, lambda a: (0, 0))
    return pl.pallas_call(
        _mlp_table_body,
        grid=(_NUM_BUCKETS,),
        in_specs=[
            const((_U_STRIDE, 32)),
            pl.BlockSpec((1, 32), lambda a: (a, 0)),
            const((1, 32)), const((1, 32)), const((1, 32)), const((1, 32)),
            const((32, 32)), const((32, 32)), const((1, 32)),
            const((1, 32)), const((1, 32)),
            const((32, 16)), const((1, 16)),
        ],
        out_specs=pl.BlockSpec((_U_STRIDE, _D_OUT), lambda a: (a, 0)),
        out_shape=jax.ShapeDtypeStruct((_NUM_BUCKETS * _U_STRIDE, _D_OUT),
                                       jnp.float32),
    )(uid_pad, age_table, s1a, t1a, s1b, t1b, w1a, w1b, b1, s2, t2, w2, b2)


@functools.partial(
    pl.kernel,
    mesh=plsc.VectorSubcoreMesh(core_axis_name="c", subcore_axis_name="s"),
    out_type=jax.ShapeDtypeStruct((_B, _D_OUT), jnp.float32),
    scratch_types=[
        pltpu.VMEM((_BPW,), jnp.int32),
        pltpu.VMEM((_BPW,), jnp.float32),
        pltpu.VMEM((_N_CHUNKS, _CHUNK), jnp.int32),
        pltpu.VMEM((_BPW, _D_OUT), jnp.float32),
        pltpu.SemaphoreType.DMA,
    ],
)
def _sc_lookup(f_hbm, uid_hbm, age_hbm, out_hbm, uid_v, age_v, idx_v, rows_v,
               sem):
    wid = lax.axis_index("s") * _NC + lax.axis_index("c")
    base = wid * _BPW
    pltpu.sync_copy(uid_hbm.at[pl.ds(base, _BPW)], uid_v)
    pltpu.sync_copy(age_hbm.at[pl.ds(base, _BPW)], age_v)
    for j in range(_N_CHUNKS):
        for i in range(_CHUNK // 16):
            off = j * _CHUNK + i * 16
            a = age_v[pl.ds(off, 16)]
            u = uid_v[pl.ds(off, 16)]
            bkt = jnp.zeros((16,), jnp.int32)
            for bound in _BOUNDS:
                bkt = bkt + (a >= bound).astype(jnp.int32)
            idx_v[j, pl.ds(i * 16, 16)] = u + bkt * _U_STRIDE
    copies = [
        pltpu.async_copy(f_hbm.at[idx_v.at[j]],
                         rows_v.at[pl.ds(j * _CHUNK, _CHUNK)], sem)
        for j in range(_N_CHUNKS)
    ]
    for c in copies:
        c.wait()
    pltpu.sync_copy(rows_v, out_hbm.at[pl.ds(base, _BPW)])


def kernel(user_id, item_id, price, user_age, user_id_table, age_table,
           bn1_gamma, bn1_beta, bn1_mean, bn1_var, W1, b1,
           bn2_gamma, bn2_beta, bn2_mean, bn2_var, W2, b2):
    del item_id, price
    s1 = bn1_gamma * jax.lax.rsqrt(bn1_var + _BN_EPS)
    t1 = bn1_beta - bn1_mean * s1
    s2 = bn2_gamma * jax.lax.rsqrt(bn2_var + _BN_EPS)
    t2 = bn2_beta - bn2_mean * s2
    uid_pad = jnp.pad(user_id_table,
                      ((0, _U_STRIDE - user_id_table.shape[0]), (0, 0)))
    row = lambda v: v.reshape(1, -1)
    f_table = _build_f_table(
        uid_pad, age_table,
        row(s1[:32]), row(t1[:32]), row(s1[32:]), row(t1[32:]),
        W1[:32], W1[32:], row(b1), row(s2), row(t2), W2, row(b2))
    return _sc_lookup(f_table, user_id, user_age)


# speedup vs baseline: 3.0279x; 3.0279x over previous
import functools

import jax
import jax.numpy as jnp
from jax import lax
from jax.experimental import pallas as pl
from jax.experimental.pallas import tpu as pltpu
from jax.experimental.pallas import tpu_sc as plsc

_BN_EPS = 1e-3
_NUM_BUCKETS = 11
_BOUNDS = tuple(float(v) for v in range(1, 100, 10))
_U_STRIDE = 128
_B = 16384
_D_OUT = 16
_NC, _NS = 2, 16
_NW = _NC * _NS
_BPW = _B // _NW
_CHUNK = 128
_N_CHUNKS = _BPW // _CHUNK


def _mlp_table_body(uid_ref, age_ref, s1a_ref, t1a_ref, s1b_ref, t1b_ref,
                    w1a_ref, w1b_ref, b1_ref, s2_ref, t2_ref, w2_ref, b2_ref,
                    f_ref):
    xu = uid_ref[...] * s1a_ref[...] + t1a_ref[...]
    hu = jnp.dot(xu, w1a_ref[...], preferred_element_type=jnp.float32)
    for a in range(_NUM_BUCKETS):
        xa = age_ref[a] * s1b_ref[...] + t1b_ref[...]
        ca = jnp.dot(xa, w1b_ref[...], preferred_element_type=jnp.float32)
        h = jnp.maximum(hu + ca + b1_ref[...], 0.0)
        h = h * s2_ref[...] + t2_ref[...]
        f = jnp.dot(h, w2_ref[...], preferred_element_type=jnp.float32)
        f_ref[pl.ds(a * _U_STRIDE, _U_STRIDE), :] = jnp.maximum(
            f + b2_ref[...], 0.0)


def _build_f_table(uid_pad, age_table, s1a, t1a, s1b, t1b, w1a, w1b, b1,
                   s2, t2, w2, b2):
    return pl.pallas_call(
        _mlp_table_body,
        out_shape=jax.ShapeDtypeStruct((_NUM_BUCKETS * _U_STRIDE, _D_OUT),
                                       jnp.float32),
    )(uid_pad, age_table, s1a, t1a, s1b, t1b, w1a, w1b, b1, s2, t2, w2, b2)


@functools.partial(
    pl.kernel,
    mesh=plsc.VectorSubcoreMesh(core_axis_name="c", subcore_axis_name="s"),
    compiler_params=pltpu.CompilerParams(use_tc_tiling_on_sc=False),
    out_type=jax.ShapeDtypeStruct((_B, _D_OUT), jnp.float32),
    scratch_types=[
        pltpu.VMEM((_BPW,), jnp.int32),
        pltpu.VMEM((_BPW,), jnp.float32),
        pltpu.VMEM((_N_CHUNKS, _CHUNK), jnp.int32),
        pltpu.VMEM((_BPW, _D_OUT), jnp.float32),
        pltpu.SemaphoreType.DMA,
        pltpu.SemaphoreType.DMA,
        pltpu.SemaphoreType.DMA,
    ],
)
def _sc_lookup(f_hbm, uid_hbm, age_hbm, out_hbm, uid_v, age_v, idx_v, rows_v,
               sem_in, sem_g, sem_out):
    wid = lax.axis_index("s") * _NC + lax.axis_index("c")
    base = wid * _BPW
    cp_u = pltpu.async_copy(uid_hbm.at[pl.ds(base, _BPW)], uid_v, sem_in)
    cp_a = pltpu.async_copy(age_hbm.at[pl.ds(base, _BPW)], age_v, sem_in)
    cp_u.wait()
    cp_a.wait()
    gathers = []
    for j in range(_N_CHUNKS):
        for i in range(_CHUNK // 16):
            off = j * _CHUNK + i * 16
            a = age_v[pl.ds(off, 16)]
            u = uid_v[pl.ds(off, 16)]
            bkt = jnp.zeros((16,), jnp.int32)
            one = jnp.ones((16,), jnp.int32)
            for bound in _BOUNDS:
                bkt = jnp.where(a >= bound, bkt + one, bkt)
            idx_v[j, pl.ds(i * 16, 16)] = u + bkt * _U_STRIDE
        gathers.append(pltpu.async_copy(
            f_hbm.at[idx_v.at[j]], rows_v.at[pl.ds(j * _CHUNK, _CHUNK)],
            sem_g))
    outs = []
    for j in range(_N_CHUNKS):
        gathers[j].wait()
        outs.append(pltpu.async_copy(
            rows_v.at[pl.ds(j * _CHUNK, _CHUNK)],
            out_hbm.at[pl.ds(base + j * _CHUNK, _CHUNK)], sem_out))
    for cp in outs:
        cp.wait()


def kernel(user_id, item_id, price, user_age, user_id_table, age_table,
           bn1_gamma, bn1_beta, bn1_mean, bn1_var, W1, b1,
           bn2_gamma, bn2_beta, bn2_mean, bn2_var, W2, b2):
    del item_id, price
    s1 = bn1_gamma * jax.lax.rsqrt(bn1_var + _BN_EPS)
    t1 = bn1_beta - bn1_mean * s1
    s2 = bn2_gamma * jax.lax.rsqrt(bn2_var + _BN_EPS)
    t2 = bn2_beta - bn2_mean * s2
    uid_pad = jnp.pad(user_id_table,
                      ((0, _U_STRIDE - user_id_table.shape[0]), (0, 0)))
    row = lambda v: v.reshape(1, -1)
    f_table = _build_f_table(
        uid_pad, age_table.reshape(_NUM_BUCKETS, 1, 32),
        row(s1[:32]), row(t1[:32]), row(s1[32:]), row(t1[32:]),
        W1[:32], W1[32:], row(b1), row(s2), row(t2), W2, row(b2))
    return _sc_lookup(f_table, user_id, user_age)
